# baseline (speedup 1.0000x reference)
import functools

import jax
import jax.numpy as jnp
from jax import lax
from jax.experimental import pallas as pl
from jax.experimental.pallas import tpu as pltpu
from jax.experimental.pallas import tpu_sc as plsc

N = 10000
E = 160000
F_IN = 128
DIM = 32
B = 128

NW = 32
CH = 128
E_PAD = 163840
EW = E_PAD // NW
N_S = 10016

_f32 = jnp.float32


def _mm(a, b):
    return lax.dot_general(a, b, (((1,), (0,)), ((), ())),
                           precision=lax.Precision.HIGHEST,
                           preferred_element_type=_f32)


def _lin0_body(x_ref, w_ref, b_ref, o_ref):
    o_ref[...] = jnp.maximum(_mm(x_ref[...], w_ref[...]) + b_ref[...], 0.0)


def _edge_h_body(ea_ref, w_ref, b_ref, o_ref):
    o_ref[...] = jnp.maximum(_mm(ea_ref[...], w_ref[...]) + b_ref[...], 0.0)


def _we_body(h_ref, w_ref, b_ref, o_ref):
    o_ref[...] = _mm(h_ref[...], w_ref[...]) + b_ref[...]


def _msg_body(g_ref, we_ref, o_ref):
    g = g_ref[...]
    acc = g[:, 0:1] * we_ref[:, 0:DIM]
    for i in range(1, DIM):
        acc += g[:, i:i + 1] * we_ref[:, i * DIM:(i + 1) * DIM]
    o_ref[...] = acc


def _update_body(aggp_ref, degp_ref, h_ref, wr_ref, br_ref, wih_ref,
                 bih_ref, whh_ref, bhh_ref, o_ref):
    agg = aggp_ref[0] + aggp_ref[1]
    deg = jnp.maximum(degp_ref[0, :, 0:1] + degp_ref[1, :, 0:1], 1.0)
    h = h_ref[...]
    m = jnp.maximum(agg / deg + _mm(h, wr_ref[...]) + br_ref[...], 0.0)
    gi = _mm(m, wih_ref[...]) + bih_ref[...]
    gh = _mm(h, whh_ref[...]) + bhh_ref[...]
    r = jax.nn.sigmoid(gi[:, 0:DIM] + gh[:, 0:DIM])
    z = jax.nn.sigmoid(gi[:, DIM:2 * DIM] + gh[:, DIM:2 * DIM])
    n = jnp.tanh(gi[:, 2 * DIM:3 * DIM] + r * gh[:, 2 * DIM:3 * DIM])
    o_ref[...] = (1.0 - z) * n + z * h


def _s2s_lstm_body(hs_ref, cs_ref, run_ref, den_ref, wc_ref, wr_ref, b_ref,
                   hso_ref, cso_ref):
    ident = (lax.broadcasted_iota(jnp.int32, (B, B), 0)
             == lax.broadcasted_iota(jnp.int32, (B, B), 1))
    dcol = jnp.sum(jnp.where(ident, den_ref[...], 0.0), axis=1, keepdims=True)
    r_norm = run_ref[...] / (dcol + 1e-16)
    hs = hs_ref[...]
    gates = _mm(hs, wc_ref[...]) + _mm(r_norm, wr_ref[...]) + b_ref[...]
    ig = jax.nn.sigmoid(gates[:, 0:DIM])
    fg = jax.nn.sigmoid(gates[:, DIM:2 * DIM])
    gg = jnp.tanh(gates[:, 2 * DIM:3 * DIM])
    og = jax.nn.sigmoid(gates[:, 3 * DIM:4 * DIM])
    cs = fg * cs_ref[...] + ig * gg
    hso_ref[...] = og * jnp.tanh(cs)
    cso_ref[...] = cs


def _s2s_pass1_body(out_ref, bat_ref, q_ref, e_ref, emax_ref, acc_ref):
    pi = pl.program_id(0)
    npr = pl.num_programs(0)
    eqb = bat_ref[...] == lax.broadcasted_iota(jnp.int32,
                                               (out_ref.shape[0], B), 1)
    qn = _mm(eqb.astype(_f32), q_ref[...])
    e_blk = jnp.sum(out_ref[...] * qn, axis=1, keepdims=True)
    e_ref[...] = e_blk
    vals = jnp.where(eqb, e_blk, -3.0e38)
    blkmax = jnp.max(vals, axis=0, keepdims=True)

    @pl.when(pi == 0)
    def _():
        acc_ref[...] = jnp.full((1, B), -3.0e38, _f32)

    acc_ref[...] = jnp.maximum(acc_ref[...], blkmax)

    @pl.when(pi == npr - 1)
    def _():
        emax_ref[...] = acc_ref[...]


def _s2s_pass2_body(out_ref, bat_ref, e_ref, emax_ref, den_ref, run_ref,
                    accd_ref, accr_ref):
    pi = pl.program_id(0)
    npr = pl.num_programs(0)
    nb = out_ref.shape[0]
    eqb = bat_ref[...] == lax.broadcasted_iota(jnp.int32, (nb, B), 1)
    emax_n = jnp.sum(jnp.where(eqb, emax_ref[...], 0.0), axis=1,
                     keepdims=True)
    eexp = jnp.exp(e_ref[...] - emax_n)
    wts = jnp.where(eqb, eexp, 0.0)

    @pl.when(pi == 0)
    def _():
        accd_ref[...] = jnp.zeros((1, B), _f32)
        accr_ref[...] = jnp.zeros((B, DIM), _f32)

    accd_ref[...] += jnp.sum(wts, axis=0, keepdims=True)
    accr_ref[...] += lax.dot_general(wts, out_ref[...],
                                     (((0,), (0,)), ((), ())),
                                     precision=lax.Precision.HIGHEST,
                                     preferred_element_type=_f32)

    @pl.when(pi == npr - 1)
    def _():
        den_ref[...] = accd_ref[...]
        run_ref[...] = accr_ref[...]


def _s2s_final_body(hs_ref, run_ref, den_ref, w1a_ref, w1b_ref, b1_ref,
                    w2_ref, b2_ref, o_ref):
    ident = (lax.broadcasted_iota(jnp.int32, (B, B), 0)
             == lax.broadcasted_iota(jnp.int32, (B, B), 1))
    dcol = jnp.sum(jnp.where(ident, den_ref[...], 0.0), axis=1, keepdims=True)
    r_norm = run_ref[...] / (dcol + 1e-16)
    o1 = jnp.maximum(_mm(hs_ref[...], w1a_ref[...]) + _mm(r_norm, w1b_ref[...])
                     + b1_ref[...], 0.0)
    o_ref[...] = jnp.sum(o1 * w2_ref[...], axis=1, keepdims=True) + b2_ref[...]


def _sc_gather_body(tab_ref, idx_ref, out_ref, idxv, rows, sem):
    wid = lax.axis_index("s") * 2 + lax.axis_index("c")
    base = wid * EW

    def step(j, carry):
        off = base + j * CH
        pltpu.sync_copy(idx_ref.at[pl.ds(off, CH)], idxv)
        pltpu.async_copy(tab_ref.at[idxv], rows, sem).wait()
        pltpu.sync_copy(rows, out_ref.at[pl.ds(off, CH)])
        return carry

    lax.fori_loop(0, EW // CH, step, 0)


def _sc_scatter_body(msg_ref, dst_ref, zero_ref, out_ref, idxv, rows, shared):
    cid = lax.axis_index("c")
    sid = lax.axis_index("s")
    wid = sid * 2 + cid

    @pl.when(sid == 0)
    def _():
        pltpu.sync_copy(zero_ref, shared)

    plsc.subcore_barrier()

    def step(j, carry):
        off = wid * EW + j * CH
        pltpu.sync_copy(dst_ref.at[pl.ds(off, CH)], idxv)
        pltpu.sync_copy(msg_ref.at[pl.ds(off, CH)], rows)
        pltpu.sync_copy(rows, shared.at[idxv], add=True)
        return carry

    lax.fori_loop(0, EW // CH, step, 0)
    plsc.subcore_barrier()

    @pl.when(sid == 0)
    def _():
        pltpu.sync_copy(shared, out_ref.at[cid])


@functools.cache
def _sc_mesh():
    return plsc.VectorSubcoreMesh(core_axis_name="c", subcore_axis_name="s")


_SC_PARAMS = pltpu.CompilerParams(use_tc_tiling_on_sc=False)


def _sc_gather(tab, idx):
    return pl.kernel(
        _sc_gather_body,
        out_type=jax.ShapeDtypeStruct((E_PAD, DIM), _f32),
        mesh=_sc_mesh(),
        compiler_params=_SC_PARAMS,
        scratch_types=[
            pltpu.VMEM((CH,), jnp.int32),
            pltpu.VMEM((CH, DIM), _f32),
            pltpu.SemaphoreType.DMA,
        ],
    )(tab, idx)


def _sc_scatter(msg, dst, zeros, width):
    return pl.kernel(
        _sc_scatter_body,
        out_type=jax.ShapeDtypeStruct((2, N_S, width), _f32),
        mesh=_sc_mesh(),
        compiler_params=_SC_PARAMS,
        scratch_types=[
            pltpu.VMEM((CH,), jnp.int32),
            pltpu.VMEM((CH, width), _f32),
            pltpu.VMEM_SHARED((N_S, width), _f32),
        ],
    )(msg, dst, zeros)


def kernel(x, edge_index, edge_attr, batch, W0, b0, We1, be1, We2, be2, Wr,
           br, Wih, Whh, bih, bhh, Wih_s, Whh_s, bih_s, bhh_s, W1, b1, W2,
           b2):
    src = edge_index[0]
    dst = edge_index[1]
    pad = E_PAD - E
    src_p = jnp.concatenate([src, jnp.zeros((pad,), jnp.int32)])
    dst_p = jnp.concatenate([dst, jnp.full((pad,), N, jnp.int32)])
    ea_p = jnp.pad(edge_attr, ((0, pad), (0, 3)))
    ones_col = jnp.concatenate(
        [jnp.ones((E, 1), _f32), jnp.zeros((pad, 1), _f32)], axis=0)
    ones16 = jnp.pad(ones_col, ((0, 0), (0, 15)))
    z32 = jnp.zeros((N_S, DIM), _f32)
    z16 = jnp.zeros((N_S, 16), _f32)
    bat2 = batch.reshape(N, 1)
    We1p = jnp.pad(We1, ((0, 3), (0, 0)))

    b0r = b0.reshape(1, DIM)
    be1r = be1.reshape(1, 128)
    be2r = be2.reshape(1, DIM * DIM)
    brr = br.reshape(1, DIM)
    bihr = bih.reshape(1, 3 * DIM)
    bhhr = bhh.reshape(1, 3 * DIM)
    Wc_s = Wih_s[0:DIM, :] + Whh_s
    Wr_s = Wih_s[DIM:2 * DIM, :]
    b_s = (bih_s + bhh_s).reshape(1, 4 * DIM)
    W1a = W1[0:DIM, :]
    W1b = W1[DIM:2 * DIM, :]
    b1r = b1.reshape(1, DIM)
    w2row = W2.reshape(1, DIM)
    b2r = b2.reshape(1, 1)

    x_p = jnp.pad(x, ((0, N_S - N), (0, 0)))
    h = pl.pallas_call(
        _lin0_body,
        out_shape=jax.ShapeDtypeStruct((N_S, DIM), _f32),
    )(x_p, W0, b0r)

    EBH = 8192
    H = pl.pallas_call(
        _edge_h_body,
        grid=(E_PAD // EBH,),
        in_specs=[
            pl.BlockSpec((EBH, 8), lambda i: (i, 0)),
            pl.BlockSpec((8, 128), lambda i: (0, 0)),
            pl.BlockSpec((1, 128), lambda i: (0, 0)),
        ],
        out_specs=pl.BlockSpec((EBH, 128), lambda i: (i, 0)),
        out_shape=jax.ShapeDtypeStruct((E_PAD, 128), _f32),
    )(ea_p, We1p, be1r)

    EBW = 2048
    We_r = pl.pallas_call(
        _we_body,
        grid=(E_PAD // EBW,),
        in_specs=[
            pl.BlockSpec((EBW, 128), lambda i: (i, 0)),
            pl.BlockSpec((128, DIM * DIM), lambda i: (0, 0)),
            pl.BlockSpec((1, DIM * DIM), lambda i: (0, 0)),
        ],
        out_specs=pl.BlockSpec((EBW, DIM * DIM), lambda i: (i, 0)),
        out_shape=jax.ShapeDtypeStruct((E_PAD, DIM * DIM), _f32),
    )(H, We2, be2r)

    degp = _sc_scatter(ones16, dst_p, z16, 16)

    EBM = 2048
    for _ in range(3):
        G = _sc_gather(h, src_p)
        msg = pl.pallas_call(
            _msg_body,
            grid=(E_PAD // EBM,),
            in_specs=[
                pl.BlockSpec((EBM, DIM), lambda i: (i, 0)),
                pl.BlockSpec((EBM, DIM * DIM), lambda i: (i, 0)),
            ],
            out_specs=pl.BlockSpec((EBM, DIM), lambda i: (i, 0)),
            out_shape=jax.ShapeDtypeStruct((E_PAD, DIM), _f32),
        )(G, We_r)
        aggp = _sc_scatter(msg, dst_p, z32, DIM)
        NBU = 2504
        h = pl.pallas_call(
            _update_body,
            grid=(N_S // NBU,),
            in_specs=[
                pl.BlockSpec((2, NBU, DIM), lambda i: (0, i, 0)),
                pl.BlockSpec((2, NBU, 16), lambda i: (0, i, 0)),
                pl.BlockSpec((NBU, DIM), lambda i: (i, 0)),
                pl.BlockSpec((DIM, DIM), lambda i: (0, 0)),
                pl.BlockSpec((1, DIM), lambda i: (0, 0)),
                pl.BlockSpec((DIM, 3 * DIM), lambda i: (0, 0)),
                pl.BlockSpec((1, 3 * DIM), lambda i: (0, 0)),
                pl.BlockSpec((DIM, 3 * DIM), lambda i: (0, 0)),
                pl.BlockSpec((1, 3 * DIM), lambda i: (0, 0)),
            ],
            out_specs=pl.BlockSpec((NBU, DIM), lambda i: (i, 0)),
            out_shape=jax.ShapeDtypeStruct((N_S, DIM), _f32),
        )(aggp, degp, h, Wr, brr, Wih, bihr, Whh, bhhr)

    h = lax.slice(h, (0, 0), (N, DIM))
    hs = jnp.zeros((B, DIM), _f32)
    cs = jnp.zeros((B, DIM), _f32)
    r_un = jnp.zeros((B, DIM), _f32)
    den = jnp.zeros((1, B), _f32)
    NB = 2000
    for _ in range(3):
        hs, cs = pl.pallas_call(
            _s2s_lstm_body,
            out_shape=(jax.ShapeDtypeStruct((B, DIM), _f32),
                       jax.ShapeDtypeStruct((B, DIM), _f32)),
        )(hs, cs, r_un, den, Wc_s, Wr_s, b_s)
        e, emax = pl.pallas_call(
            _s2s_pass1_body,
            grid=(N // NB,),
            in_specs=[
                pl.BlockSpec((NB, DIM), lambda i: (i, 0)),
                pl.BlockSpec((NB, 1), lambda i: (i, 0)),
                pl.BlockSpec((B, DIM), lambda i: (0, 0)),
            ],
            out_specs=(pl.BlockSpec((NB, 1), lambda i: (i, 0)),
                       pl.BlockSpec((1, B), lambda i: (0, 0))),
            out_shape=(jax.ShapeDtypeStruct((N, 1), _f32),
                       jax.ShapeDtypeStruct((1, B), _f32)),
            scratch_shapes=[pltpu.VMEM((1, B), _f32)],
        )(h, bat2, hs)
        den, r_un = pl.pallas_call(
            _s2s_pass2_body,
            grid=(N // NB,),
            in_specs=[
                pl.BlockSpec((NB, DIM), lambda i: (i, 0)),
                pl.BlockSpec((NB, 1), lambda i: (i, 0)),
                pl.BlockSpec((NB, 1), lambda i: (i, 0)),
                pl.BlockSpec((1, B), lambda i: (0, 0)),
            ],
            out_specs=(pl.BlockSpec((1, B), lambda i: (0, 0)),
                       pl.BlockSpec((B, DIM), lambda i: (0, 0))),
            out_shape=(jax.ShapeDtypeStruct((1, B), _f32),
                       jax.ShapeDtypeStruct((B, DIM), _f32)),
            scratch_shapes=[pltpu.VMEM((1, B), _f32),
                            pltpu.VMEM((B, DIM), _f32)],
        )(h, bat2, e, emax)

    o = pl.pallas_call(
        _s2s_final_body,
        out_shape=jax.ShapeDtypeStruct((B, 1), _f32),
    )(hs, r_un, den, W1a, W1b, b1r, w2row, b2r)
    return o.reshape(-1)


# speedup vs baseline: 2.0763x; 2.0763x over previous
import functools

import jax
import jax.numpy as jnp
from jax import lax
from jax.experimental import pallas as pl
from jax.experimental.pallas import tpu as pltpu
from jax.experimental.pallas import tpu_sc as plsc

N = 10000
E = 160000
F_IN = 128
DIM = 32
B = 128

NW = 32
CH = 128
E_PAD = 163840
EW = E_PAD // NW
N_S = 10016

_f32 = jnp.float32


def _mm(a, b):
    return lax.dot_general(a, b, (((1,), (0,)), ((), ())),
                           precision=lax.Precision.HIGHEST,
                           preferred_element_type=_f32)


def _lin0_body(x_ref, w_ref, b_ref, o_ref):
    o_ref[...] = jnp.maximum(_mm(x_ref[...], w_ref[...]) + b_ref[...], 0.0)


def _edge_ht_body(wt_ref, ea_ref, b_ref, o_ref):
    o_ref[...] = jnp.maximum(_mm(wt_ref[...], ea_ref[...]) + b_ref[...], 0.0)


def _wet_body(w2t_ref, ht_ref, b_ref, o_ref):
    o_ref[...] = _mm(w2t_ref[...], ht_ref[...]) + b_ref[...]


def _msg_body(g_ref, wet_ref, o_ref):
    gt = jnp.transpose(g_ref[...])
    acc = gt[0:1, :] * wet_ref[0:DIM, :]
    for i in range(1, DIM):
        acc += gt[i:i + 1, :] * wet_ref[i * DIM:(i + 1) * DIM, :]
    o_ref[...] = jnp.transpose(acc)


def _update_body(aggp_ref, degp_ref, h_ref, wr_ref, br_ref, wih_ref,
                 bih_ref, whh_ref, bhh_ref, o_ref):
    agg = aggp_ref[0] + aggp_ref[1]
    deg = jnp.maximum(degp_ref[0, :, 0:1] + degp_ref[1, :, 0:1], 1.0)
    h = h_ref[...]
    m = jnp.maximum(agg / deg + _mm(h, wr_ref[...]) + br_ref[...], 0.0)
    gi = _mm(m, wih_ref[...]) + bih_ref[...]
    gh = _mm(h, whh_ref[...]) + bhh_ref[...]
    r = jax.nn.sigmoid(gi[:, 0:DIM] + gh[:, 0:DIM])
    z = jax.nn.sigmoid(gi[:, DIM:2 * DIM] + gh[:, DIM:2 * DIM])
    n = jnp.tanh(gi[:, 2 * DIM:3 * DIM] + r * gh[:, 2 * DIM:3 * DIM])
    o_ref[...] = (1.0 - z) * n + z * h


def _s2s_lstm_body(hs_ref, cs_ref, run_ref, den_ref, wc_ref, wr_ref, b_ref,
                   hso_ref, cso_ref):
    ident = (lax.broadcasted_iota(jnp.int32, (B, B), 0)
             == lax.broadcasted_iota(jnp.int32, (B, B), 1))
    dcol = jnp.sum(jnp.where(ident, den_ref[...], 0.0), axis=1, keepdims=True)
    r_norm = run_ref[...] / (dcol + 1e-16)
    hs = hs_ref[...]
    gates = _mm(hs, wc_ref[...]) + _mm(r_norm, wr_ref[...]) + b_ref[...]
    ig = jax.nn.sigmoid(gates[:, 0:DIM])
    fg = jax.nn.sigmoid(gates[:, DIM:2 * DIM])
    gg = jnp.tanh(gates[:, 2 * DIM:3 * DIM])
    og = jax.nn.sigmoid(gates[:, 3 * DIM:4 * DIM])
    cs = fg * cs_ref[...] + ig * gg
    hso_ref[...] = og * jnp.tanh(cs)
    cso_ref[...] = cs


def _s2s_pass1_body(out_ref, bat_ref, q_ref, e_ref, emax_ref, acc_ref):
    pi = pl.program_id(0)
    npr = pl.num_programs(0)
    eqb = bat_ref[...] == lax.broadcasted_iota(jnp.int32,
                                               (out_ref.shape[0], B), 1)
    qn = _mm(eqb.astype(_f32), q_ref[...])
    e_blk = jnp.sum(out_ref[...] * qn, axis=1, keepdims=True)
    e_ref[...] = e_blk
    vals = jnp.where(eqb, e_blk, -3.0e38)
    blkmax = jnp.max(vals, axis=0, keepdims=True)

    @pl.when(pi == 0)
    def _():
        acc_ref[...] = jnp.full((1, B), -3.0e38, _f32)

    acc_ref[...] = jnp.maximum(acc_ref[...], blkmax)

    @pl.when(pi == npr - 1)
    def _():
        emax_ref[...] = acc_ref[...]


def _s2s_pass2_body(out_ref, bat_ref, e_ref, emax_ref, den_ref, run_ref,
                    accd_ref, accr_ref):
    pi = pl.program_id(0)
    npr = pl.num_programs(0)
    nb = out_ref.shape[0]
    eqb = bat_ref[...] == lax.broadcasted_iota(jnp.int32, (nb, B), 1)
    emax_n = jnp.sum(jnp.where(eqb, emax_ref[...], 0.0), axis=1,
                     keepdims=True)
    eexp = jnp.exp(e_ref[...] - emax_n)
    wts = jnp.where(eqb, eexp, 0.0)

    @pl.when(pi == 0)
    def _():
        accd_ref[...] = jnp.zeros((1, B), _f32)
        accr_ref[...] = jnp.zeros((B, DIM), _f32)

    accd_ref[...] += jnp.sum(wts, axis=0, keepdims=True)
    accr_ref[...] += lax.dot_general(wts, out_ref[...],
                                     (((0,), (0,)), ((), ())),
                                     precision=lax.Precision.HIGHEST,
                                     preferred_element_type=_f32)

    @pl.when(pi == npr - 1)
    def _():
        den_ref[...] = accd_ref[...]
        run_ref[...] = accr_ref[...]


def _s2s_final_body(hs_ref, run_ref, den_ref, w1a_ref, w1b_ref, b1_ref,
                    w2_ref, b2_ref, o_ref):
    ident = (lax.broadcasted_iota(jnp.int32, (B, B), 0)
             == lax.broadcasted_iota(jnp.int32, (B, B), 1))
    dcol = jnp.sum(jnp.where(ident, den_ref[...], 0.0), axis=1, keepdims=True)
    r_norm = run_ref[...] / (dcol + 1e-16)
    o1 = jnp.maximum(_mm(hs_ref[...], w1a_ref[...]) + _mm(r_norm, w1b_ref[...])
                     + b1_ref[...], 0.0)
    o_ref[...] = jnp.sum(o1 * w2_ref[...], axis=1, keepdims=True) + b2_ref[...]


def _sc_gather_body(tab_ref, idx_ref, out_ref, idxv, rows, sem):
    wid = lax.axis_index("s") * 2 + lax.axis_index("c")
    base = wid * EW

    def step(j, carry):
        off = base + j * CH
        pltpu.sync_copy(idx_ref.at[pl.ds(off, CH)], idxv)
        pltpu.async_copy(tab_ref.at[idxv], rows, sem).wait()
        pltpu.sync_copy(rows, out_ref.at[pl.ds(off, CH)])
        return carry

    lax.fori_loop(0, EW // CH, step, 0)


def _sc_scatter_body(msg_ref, dst_ref, zero_ref, out_ref, idxv, rows, shared):
    cid = lax.axis_index("c")
    sid = lax.axis_index("s")
    wid = sid * 2 + cid

    @pl.when(sid == 0)
    def _():
        pltpu.sync_copy(zero_ref, shared)

    plsc.subcore_barrier()

    def step(j, carry):
        off = wid * EW + j * CH
        pltpu.sync_copy(dst_ref.at[pl.ds(off, CH)], idxv)
        pltpu.sync_copy(msg_ref.at[pl.ds(off, CH)], rows)
        pltpu.sync_copy(rows, shared.at[idxv], add=True)
        return carry

    lax.fori_loop(0, EW // CH, step, 0)
    plsc.subcore_barrier()

    @pl.when(sid == 0)
    def _():
        pltpu.sync_copy(shared, out_ref.at[cid])


@functools.cache
def _sc_mesh():
    return plsc.VectorSubcoreMesh(core_axis_name="c", subcore_axis_name="s")


_SC_PARAMS = pltpu.CompilerParams(use_tc_tiling_on_sc=False)


def _sc_gather(tab, idx):
    return pl.kernel(
        _sc_gather_body,
        out_type=jax.ShapeDtypeStruct((E_PAD, DIM), _f32),
        mesh=_sc_mesh(),
        compiler_params=_SC_PARAMS,
        scratch_types=[
            pltpu.VMEM((CH,), jnp.int32),
            pltpu.VMEM((CH, DIM), _f32),
            pltpu.SemaphoreType.DMA,
        ],
    )(tab, idx)


def _sc_scatter(msg, dst, zeros, width):
    return pl.kernel(
        _sc_scatter_body,
        out_type=jax.ShapeDtypeStruct((2, N_S, width), _f32),
        mesh=_sc_mesh(),
        compiler_params=_SC_PARAMS,
        scratch_types=[
            pltpu.VMEM((CH,), jnp.int32),
            pltpu.VMEM((CH, width), _f32),
            pltpu.VMEM_SHARED((N_S, width), _f32),
        ],
    )(msg, dst, zeros)


def kernel(x, edge_index, edge_attr, batch, W0, b0, We1, be1, We2, be2, Wr,
           br, Wih, Whh, bih, bhh, Wih_s, Whh_s, bih_s, bhh_s, W1, b1, W2,
           b2):
    src = edge_index[0]
    dst = edge_index[1]
    pad = E_PAD - E
    src_p = jnp.concatenate([src, jnp.zeros((pad,), jnp.int32)])
    dst_p = jnp.concatenate([dst, jnp.full((pad,), N, jnp.int32)])
    ea_t = jnp.pad(edge_attr.T, ((0, 3), (0, pad)))
    ones_col = jnp.concatenate(
        [jnp.ones((E, 1), _f32), jnp.zeros((pad, 1), _f32)], axis=0)
    ones16 = jnp.pad(ones_col, ((0, 0), (0, 15)))
    z32 = jnp.zeros((N_S, DIM), _f32)
    z16 = jnp.zeros((N_S, 16), _f32)
    bat2 = batch.reshape(N, 1)
    We1t = jnp.pad(We1.T, ((0, 0), (0, 3)))
    We2t = We2.T

    b0r = b0.reshape(1, DIM)
    be1c = be1.reshape(128, 1)
    be2c = be2.reshape(DIM * DIM, 1)
    brr = br.reshape(1, DIM)
    bihr = bih.reshape(1, 3 * DIM)
    bhhr = bhh.reshape(1, 3 * DIM)
    Wc_s = Wih_s[0:DIM, :] + Whh_s
    Wr_s = Wih_s[DIM:2 * DIM, :]
    b_s = (bih_s + bhh_s).reshape(1, 4 * DIM)
    W1a = W1[0:DIM, :]
    W1b = W1[DIM:2 * DIM, :]
    b1r = b1.reshape(1, DIM)
    w2row = W2.reshape(1, DIM)
    b2r = b2.reshape(1, 1)

    x_p = jnp.pad(x, ((0, N_S - N), (0, 0)))
    h = pl.pallas_call(
        _lin0_body,
        out_shape=jax.ShapeDtypeStruct((N_S, DIM), _f32),
    )(x_p, W0, b0r)

    EBH = 8192
    Ht = pl.pallas_call(
        _edge_ht_body,
        grid=(E_PAD // EBH,),
        in_specs=[
            pl.BlockSpec((128, 8), lambda i: (0, 0)),
            pl.BlockSpec((8, EBH), lambda i: (0, i)),
            pl.BlockSpec((128, 1), lambda i: (0, 0)),
        ],
        out_specs=pl.BlockSpec((128, EBH), lambda i: (0, i)),
        out_shape=jax.ShapeDtypeStruct((128, E_PAD), _f32),
    )(We1t, ea_t, be1c)

    EBW = 512
    WeT = pl.pallas_call(
        _wet_body,
        grid=(E_PAD // EBW,),
        in_specs=[
            pl.BlockSpec((DIM * DIM, 128), lambda i: (0, 0)),
            pl.BlockSpec((128, EBW), lambda i: (0, i)),
            pl.BlockSpec((DIM * DIM, 1), lambda i: (0, 0)),
        ],
        out_specs=pl.BlockSpec((DIM * DIM, EBW), lambda i: (0, i)),
        out_shape=jax.ShapeDtypeStruct((DIM * DIM, E_PAD), _f32),
    )(We2t, Ht, be2c)

    degp = _sc_scatter(ones16, dst_p, z16, 16)

    EBM = 512
    for _ in range(3):
        G = _sc_gather(h, src_p)
        msg = pl.pallas_call(
            _msg_body,
            grid=(E_PAD // EBM,),
            in_specs=[
                pl.BlockSpec((EBM, DIM), lambda i: (i, 0)),
                pl.BlockSpec((DIM * DIM, EBM), lambda i: (0, i)),
            ],
            out_specs=pl.BlockSpec((EBM, DIM), lambda i: (i, 0)),
            out_shape=jax.ShapeDtypeStruct((E_PAD, DIM), _f32),
        )(G, WeT)
        aggp = _sc_scatter(msg, dst_p, z32, DIM)
        NBU = 2504
        h = pl.pallas_call(
            _update_body,
            grid=(N_S // NBU,),
            in_specs=[
                pl.BlockSpec((2, NBU, DIM), lambda i: (0, i, 0)),
                pl.BlockSpec((2, NBU, 16), lambda i: (0, i, 0)),
                pl.BlockSpec((NBU, DIM), lambda i: (i, 0)),
                pl.BlockSpec((DIM, DIM), lambda i: (0, 0)),
                pl.BlockSpec((1, DIM), lambda i: (0, 0)),
                pl.BlockSpec((DIM, 3 * DIM), lambda i: (0, 0)),
                pl.BlockSpec((1, 3 * DIM), lambda i: (0, 0)),
                pl.BlockSpec((DIM, 3 * DIM), lambda i: (0, 0)),
                pl.BlockSpec((1, 3 * DIM), lambda i: (0, 0)),
            ],
            out_specs=pl.BlockSpec((NBU, DIM), lambda i: (i, 0)),
            out_shape=jax.ShapeDtypeStruct((N_S, DIM), _f32),
        )(aggp, degp, h, Wr, brr, Wih, bihr, Whh, bhhr)

    h = lax.slice(h, (0, 0), (N, DIM))
    hs = jnp.zeros((B, DIM), _f32)
    cs = jnp.zeros((B, DIM), _f32)
    r_un = jnp.zeros((B, DIM), _f32)
    den = jnp.zeros((1, B), _f32)
    NB = 2000
    for _ in range(3):
        hs, cs = pl.pallas_call(
            _s2s_lstm_body,
            out_shape=(jax.ShapeDtypeStruct((B, DIM), _f32),
                       jax.ShapeDtypeStruct((B, DIM), _f32)),
        )(hs, cs, r_un, den, Wc_s, Wr_s, b_s)
        e, emax = pl.pallas_call(
            _s2s_pass1_body,
            grid=(N // NB,),
            in_specs=[
                pl.BlockSpec((NB, DIM), lambda i: (i, 0)),
                pl.BlockSpec((NB, 1), lambda i: (i, 0)),
                pl.BlockSpec((B, DIM), lambda i: (0, 0)),
            ],
            out_specs=(pl.BlockSpec((NB, 1), lambda i: (i, 0)),
                       pl.BlockSpec((1, B), lambda i: (0, 0))),
            out_shape=(jax.ShapeDtypeStruct((N, 1), _f32),
                       jax.ShapeDtypeStruct((1, B), _f32)),
            scratch_shapes=[pltpu.VMEM((1, B), _f32)],
        )(h, bat2, hs)
        den, r_un = pl.pallas_call(
            _s2s_pass2_body,
            grid=(N // NB,),
            in_specs=[
                pl.BlockSpec((NB, DIM), lambda i: (i, 0)),
                pl.BlockSpec((NB, 1), lambda i: (i, 0)),
                pl.BlockSpec((NB, 1), lambda i: (i, 0)),
                pl.BlockSpec((1, B), lambda i: (0, 0)),
            ],
            out_specs=(pl.BlockSpec((1, B), lambda i: (0, 0)),
                       pl.BlockSpec((B, DIM), lambda i: (0, 0))),
            out_shape=(jax.ShapeDtypeStruct((1, B), _f32),
                       jax.ShapeDtypeStruct((B, DIM), _f32)),
            scratch_shapes=[pltpu.VMEM((1, B), _f32),
                            pltpu.VMEM((B, DIM), _f32)],
        )(h, bat2, e, emax)

    o = pl.pallas_call(
        _s2s_final_body,
        out_shape=jax.ShapeDtypeStruct((B, 1), _f32),
    )(hs, r_un, den, W1a, W1b, b1r, w2row, b2r)
    return o.reshape(-1)
